# baseline (speedup 1.0000x reference)
import functools

import jax
import jax.numpy as jnp
from jax import lax
from jax.experimental import pallas as pl
from jax.experimental.pallas import tpu as pltpu
from jax.experimental.pallas import tpu_sc as plsc

V = 1000000
D = 64
H = 128
B = 1024
T = 50

NC = 2
NS = 16
NW = NC * NS
N = B * T
BPW = N // NW
CH = 80
NCH = BPW // CH


def _sc_gather(table, idx3):
    mesh = plsc.VectorSubcoreMesh(core_axis_name="c", subcore_axis_name="s")

    @functools.partial(
        pl.kernel,
        mesh=mesh,
        out_type=jax.ShapeDtypeStruct((N, D), jnp.float32),
        scratch_types=[
            pltpu.VMEM((NCH, CH), jnp.int32),
            pltpu.VMEM((CH, D), jnp.float32),
            pltpu.SemaphoreType.DMA,
        ],
    )
    def gather_kernel(idx_hbm, table_hbm, out_hbm, idx_v, rows_v, sem):
        wid = lax.axis_index("s") * NC + lax.axis_index("c")
        base = wid * BPW
        pltpu.sync_copy(idx_hbm.at[wid], idx_v)

        def chunk(j):
            for k in range(CH // 16):
                vec = idx_v[j, pl.ds(k * 16, 16)]
                for l in range(16):
                    pltpu.make_async_copy(
                        table_hbm.at[pl.ds(vec[l], 1)],
                        rows_v.at[pl.ds(k * 16 + l, 1)],
                        sem,
                    ).start()
            pltpu.make_async_copy(
                table_hbm.at[pl.ds(0, CH)], rows_v, sem
            ).wait()
            pltpu.sync_copy(rows_v, out_hbm.at[pl.ds(base + j * CH, CH)])

        pl.loop(0, NCH)(chunk)

    return gather_kernel(idx3, table)


def _sc_transpose(out_t):
    in2 = out_t.reshape(T * B, H)
    mesh = plsc.VectorSubcoreMesh(core_axis_name="c", subcore_axis_name="s")

    @functools.partial(
        pl.kernel,
        mesh=mesh,
        out_type=jax.ShapeDtypeStruct((N, H), jnp.float32),
        scratch_types=[
            pltpu.VMEM((NCH, CH), jnp.int32),
            pltpu.VMEM((CH, H), jnp.float32),
            pltpu.SemaphoreType.DMA,
        ],
    )
    def transpose_kernel(in_hbm, out_hbm, idx_v, rows_v, sem):
        wid = lax.axis_index("s") * NC + lax.axis_index("c")
        base = wid * BPW
        lane = lax.iota(jnp.int32, 16)

        def gen(ch):
            for k in range(CH // 16):
                iv = base + ch * CH + k * 16 + lane
                bq = ((iv.astype(jnp.float32) + 0.5) *
                      jnp.float32(1.0 / T)).astype(jnp.int32)
                tr = iv - bq * T
                idx_v[ch, pl.ds(k * 16, 16)] = tr * B + bq

        pl.loop(0, NCH)(gen)

        def chunk(ch):
            c = pltpu.make_async_copy(in_hbm.at[idx_v.at[ch]], rows_v, sem)
            c.start()
            c.wait()
            pltpu.sync_copy(rows_v, out_hbm.at[pl.ds(base + ch * CH, CH)])

        pl.loop(0, NCH)(chunk)

    return transpose_kernel(in2).reshape(B, T, H)


def _lstm_body(emb_ref, h0_ref, c0_ref, w_ref, u_ref, b_ref,
               out_ref, hf_ref, cf_ref):
    t = pl.program_id(0)

    @pl.when(t == 0)
    def _():
        hf_ref[...] = h0_ref[...]
        cf_ref[...] = c0_ref[...]

    x = emb_ref[0]
    h = hf_ref[...]
    c = cf_ref[...]
    z = (jnp.dot(x, w_ref[...], preferred_element_type=jnp.float32)
         + jnp.dot(h, u_ref[...], preferred_element_type=jnp.float32)
         + b_ref[...])
    i = jax.nn.sigmoid(z[:, 0:H])
    f = jax.nn.sigmoid(z[:, H:2 * H])
    g = jnp.tanh(z[:, 2 * H:3 * H])
    o = jax.nn.sigmoid(z[:, 3 * H:4 * H])
    c_new = f * c + i * g
    h_new = o * jnp.tanh(c_new)
    hf_ref[...] = h_new
    cf_ref[...] = c_new
    out_ref[0] = h_new


def _tc_lstm(embed, state_h, state_c, w, u, b2):
    out = pl.pallas_call(
        _lstm_body,
        grid=(T,),
        in_specs=[
            pl.BlockSpec((1, B, D), lambda t: (t, 0, 0)),
            pl.BlockSpec((B, H), lambda t: (0, 0)),
            pl.BlockSpec((B, H), lambda t: (0, 0)),
            pl.BlockSpec((D, 4 * H), lambda t: (0, 0)),
            pl.BlockSpec((H, 4 * H), lambda t: (0, 0)),
            pl.BlockSpec((1, 4 * H), lambda t: (0, 0)),
        ],
        out_specs=[
            pl.BlockSpec((1, B, H), lambda t: (t, 0, 0)),
            pl.BlockSpec((B, H), lambda t: (0, 0)),
            pl.BlockSpec((B, H), lambda t: (0, 0)),
        ],
        out_shape=[
            jax.ShapeDtypeStruct((T, B, H), jnp.float32),
            jax.ShapeDtypeStruct((B, H), jnp.float32),
            jax.ShapeDtypeStruct((B, H), jnp.float32),
        ],
    )(embed, state_h, state_c, w, u, b2)
    return out[0], out[1], out[2]


def kernel(sequence, state_h, state_c, embedding, W, U, b):
    idx3 = sequence.astype(jnp.int32).T.reshape(NW, NCH, CH)
    embed = _sc_gather(embedding, idx3).reshape(T, B, D)
    out_t, h_fin, c_fin = _tc_lstm(embed, state_h, state_c, W, U,
                                   b.reshape(1, 4 * H))
    return (_sc_transpose(out_t), h_fin, c_fin)


# speedup vs baseline: 1.0340x; 1.0340x over previous
import functools

import jax
import jax.numpy as jnp
from jax import lax
from jax.experimental import pallas as pl
from jax.experimental.pallas import tpu as pltpu
from jax.experimental.pallas import tpu_sc as plsc

V = 1000000
D = 64
H = 128
B = 1024
T = 50

NC = 2
NS = 16
NW = NC * NS
N = B * T
BPW = N // NW
CH = 80
NCH = BPW // CH


def _sc_gather(table, idx3):
    mesh = plsc.VectorSubcoreMesh(core_axis_name="c", subcore_axis_name="s")

    @functools.partial(
        pl.kernel,
        mesh=mesh,
        out_type=jax.ShapeDtypeStruct((N, D), jnp.float32),
        scratch_types=[
            pltpu.VMEM((NCH, CH), jnp.int32),
            pltpu.VMEM((CH, D), jnp.float32),
            pltpu.SemaphoreType.DMA,
        ],
    )
    def gather_kernel(idx_hbm, table_hbm, out_hbm, idx_v, rows_v, sem):
        wid = lax.axis_index("s") * NC + lax.axis_index("c")
        base = wid * BPW
        pltpu.sync_copy(idx_hbm.at[wid], idx_v)

        def chunk(j):
            for k in range(CH // 16):
                vec = idx_v[j, pl.ds(k * 16, 16)]
                for l in range(16):
                    pltpu.make_async_copy(
                        table_hbm.at[pl.ds(vec[l], 1)],
                        rows_v.at[pl.ds(k * 16 + l, 1)],
                        sem,
                    ).start()
            pltpu.make_async_copy(
                table_hbm.at[pl.ds(0, CH)], rows_v, sem
            ).wait()
            pltpu.sync_copy(rows_v, out_hbm.at[pl.ds(base + j * CH, CH)])

        pl.loop(0, NCH)(chunk)

    return gather_kernel(idx3, table)


BPB = B // NW
TP = 64


def _sc_transpose(out_t):
    in2 = out_t.reshape(T * B, H)
    mesh = plsc.VectorSubcoreMesh(core_axis_name="c", subcore_axis_name="s")

    @functools.partial(
        pl.kernel,
        mesh=mesh,
        out_type=jax.ShapeDtypeStruct((B, T, H), jnp.float32),
        scratch_types=[
            pltpu.VMEM((BPB, TP), jnp.int32),
            pltpu.VMEM((2, TP, H), jnp.float32),
            pltpu.SemaphoreType.DMA,
        ],
    )
    def transpose_kernel(in_hbm, out_hbm, idx_v, rows_v, sem):
        wid = lax.axis_index("s") * NC + lax.axis_index("c")
        b0 = wid * BPB
        lane = lax.iota(jnp.int32, 16)

        def gen(c):
            for k in range(TP // 16):
                t = jnp.minimum(k * 16 + lane, T - 1)
                idx_v[c, pl.ds(k * 16, 16)] = t * B + (b0 + c)

        pl.loop(0, BPB)(gen)

        def start(c, buf):
            pltpu.make_async_copy(
                in_hbm.at[idx_v.at[c]], rows_v.at[buf], sem).start()

        def drain_write(c, buf):
            pltpu.make_async_copy(
                in_hbm.at[idx_v.at[c]], rows_v.at[buf], sem).wait()
            pltpu.sync_copy(rows_v.at[buf, pl.ds(0, T)],
                            out_hbm.at[b0 + c])

        start(0, 0)

        def pair(c):
            start(c + 1, 1)
            drain_write(c, 0)

            @pl.when(c + 2 < BPB)
            def _():
                start(c + 2, 0)

            drain_write(c + 1, 1)

        pl.loop(0, BPB, step=2)(pair)

    return transpose_kernel(in2)


def _lstm_body(emb_ref, h0_ref, c0_ref, w_ref, u_ref, b_ref,
               out_ref, hf_ref, cf_ref):
    t = pl.program_id(0)

    @pl.when(t == 0)
    def _():
        hf_ref[...] = h0_ref[...]
        cf_ref[...] = c0_ref[...]

    x = emb_ref[0]
    h = hf_ref[...]
    c = cf_ref[...]
    z = (jnp.dot(x, w_ref[...], preferred_element_type=jnp.float32)
         + jnp.dot(h, u_ref[...], preferred_element_type=jnp.float32)
         + b_ref[...])
    i = jax.nn.sigmoid(z[:, 0:H])
    f = jax.nn.sigmoid(z[:, H:2 * H])
    g = jnp.tanh(z[:, 2 * H:3 * H])
    o = jax.nn.sigmoid(z[:, 3 * H:4 * H])
    c_new = f * c + i * g
    h_new = o * jnp.tanh(c_new)
    hf_ref[...] = h_new
    cf_ref[...] = c_new
    out_ref[0] = h_new


def _tc_lstm(embed, state_h, state_c, w, u, b2):
    out = pl.pallas_call(
        _lstm_body,
        grid=(T,),
        in_specs=[
            pl.BlockSpec((1, B, D), lambda t: (t, 0, 0)),
            pl.BlockSpec((B, H), lambda t: (0, 0)),
            pl.BlockSpec((B, H), lambda t: (0, 0)),
            pl.BlockSpec((D, 4 * H), lambda t: (0, 0)),
            pl.BlockSpec((H, 4 * H), lambda t: (0, 0)),
            pl.BlockSpec((1, 4 * H), lambda t: (0, 0)),
        ],
        out_specs=[
            pl.BlockSpec((1, B, H), lambda t: (t, 0, 0)),
            pl.BlockSpec((B, H), lambda t: (0, 0)),
            pl.BlockSpec((B, H), lambda t: (0, 0)),
        ],
        out_shape=[
            jax.ShapeDtypeStruct((T, B, H), jnp.float32),
            jax.ShapeDtypeStruct((B, H), jnp.float32),
            jax.ShapeDtypeStruct((B, H), jnp.float32),
        ],
    )(embed, state_h, state_c, w, u, b2)
    return out[0], out[1], out[2]


def kernel(sequence, state_h, state_c, embedding, W, U, b):
    idx3 = sequence.astype(jnp.int32).T.reshape(NW, NCH, CH)
    embed = _sc_gather(embedding, idx3).reshape(T, B, D)
    out_t, h_fin, c_fin = _tc_lstm(embed, state_h, state_c, W, U,
                                   b.reshape(1, 4 * H))
    return (_sc_transpose(out_t), h_fin, c_fin)


# speedup vs baseline: 1.1109x; 1.0744x over previous
import functools

import jax
import jax.numpy as jnp
from jax import lax
from jax.experimental import pallas as pl
from jax.experimental.pallas import tpu as pltpu
from jax.experimental.pallas import tpu_sc as plsc

V = 1000000
D = 64
H = 128
B = 1024
T = 50

NC = 2
NS = 16
NW = NC * NS
N = B * T
BPW = N // NW
CH = 80
NCH = BPW // CH


def _sc_gather(table, idx3):
    mesh = plsc.VectorSubcoreMesh(core_axis_name="c", subcore_axis_name="s")

    @functools.partial(
        pl.kernel,
        mesh=mesh,
        out_type=jax.ShapeDtypeStruct((N, D), jnp.float32),
        scratch_types=[
            pltpu.VMEM((NCH, CH), jnp.int32),
            pltpu.VMEM((CH, D), jnp.float32),
            pltpu.SemaphoreType.DMA,
        ],
    )
    def gather_kernel(idx_hbm, table_hbm, out_hbm, idx_v, rows_v, sem):
        wid = lax.axis_index("s") * NC + lax.axis_index("c")
        base = wid * BPW
        pltpu.sync_copy(idx_hbm.at[wid], idx_v)

        def chunk(j):
            for k in range(CH // 16):
                vec = idx_v[j, pl.ds(k * 16, 16)]
                for l in range(16):
                    pltpu.make_async_copy(
                        table_hbm.at[pl.ds(vec[l], 1)],
                        rows_v.at[pl.ds(k * 16 + l, 1)],
                        sem,
                    ).start()
            pltpu.make_async_copy(
                table_hbm.at[pl.ds(0, CH)], rows_v, sem
            ).wait()
            pltpu.sync_copy(rows_v, out_hbm.at[pl.ds(base + j * CH, CH)])

        pl.loop(0, NCH)(chunk)

    return gather_kernel(idx3, table)


def _lstm_body(emb_ref, h0_ref, c0_ref, w_ref, u_ref, b_ref,
               out_ref, hf_ref, cf_ref):
    t = pl.program_id(0)

    @pl.when(t == 0)
    def _():
        hf_ref[...] = h0_ref[...]
        cf_ref[...] = c0_ref[...]

    x = emb_ref[0]
    h = hf_ref[...]
    c = cf_ref[...]
    z = (jnp.dot(x, w_ref[...], preferred_element_type=jnp.float32)
         + jnp.dot(h, u_ref[...], preferred_element_type=jnp.float32)
         + b_ref[...])
    i = jax.nn.sigmoid(z[:, 0:H])
    f = jax.nn.sigmoid(z[:, H:2 * H])
    g = jnp.tanh(z[:, 2 * H:3 * H])
    o = jax.nn.sigmoid(z[:, 3 * H:4 * H])
    c_new = f * c + i * g
    h_new = o * jnp.tanh(c_new)
    hf_ref[...] = h_new
    cf_ref[...] = c_new
    out_ref[:, t] = h_new


def _tc_lstm(embed, state_h, state_c, w, u, b2):
    out = pl.pallas_call(
        _lstm_body,
        grid=(T,),
        in_specs=[
            pl.BlockSpec((1, B, D), lambda t: (t, 0, 0)),
            pl.BlockSpec((B, H), lambda t: (0, 0)),
            pl.BlockSpec((B, H), lambda t: (0, 0)),
            pl.BlockSpec((D, 4 * H), lambda t: (0, 0)),
            pl.BlockSpec((H, 4 * H), lambda t: (0, 0)),
            pl.BlockSpec((1, 4 * H), lambda t: (0, 0)),
        ],
        out_specs=[
            pl.BlockSpec((B, T, H), lambda t: (0, 0, 0)),
            pl.BlockSpec((B, H), lambda t: (0, 0)),
            pl.BlockSpec((B, H), lambda t: (0, 0)),
        ],
        out_shape=[
            jax.ShapeDtypeStruct((B, T, H), jnp.float32),
            jax.ShapeDtypeStruct((B, H), jnp.float32),
            jax.ShapeDtypeStruct((B, H), jnp.float32),
        ],
    )(embed, state_h, state_c, w, u, b2)
    return out[0], out[1], out[2]


def kernel(sequence, state_h, state_c, embedding, W, U, b):
    idx3 = sequence.astype(jnp.int32).T.reshape(NW, NCH, CH)
    embed = _sc_gather(embedding, idx3).reshape(T, B, D)
    out_bt, h_fin, c_fin = _tc_lstm(embed, state_h, state_c, W, U,
                                    b.reshape(1, 4 * H))
    return (out_bt, h_fin, c_fin)


# speedup vs baseline: 1.1833x; 1.0651x over previous
import functools

import jax
import jax.numpy as jnp
from jax import lax
from jax.experimental import pallas as pl
from jax.experimental.pallas import tpu as pltpu
from jax.experimental.pallas import tpu_sc as plsc

V = 1000000
D = 64
H = 128
B = 1024
T = 50

NC = 2
NS = 16
NW = NC * NS
N = B * T
BPW = N // NW
CH = 80
NCH = BPW // CH


def _sc_gather(table, idx3):
    nch = idx3.shape[1]
    bpw = nch * CH
    mesh = plsc.VectorSubcoreMesh(core_axis_name="c", subcore_axis_name="s")

    @functools.partial(
        pl.kernel,
        mesh=mesh,
        out_type=jax.ShapeDtypeStruct((NW * bpw, D), jnp.float32),
        scratch_types=[
            pltpu.VMEM((nch, CH), jnp.int32),
            pltpu.VMEM((CH, D), jnp.float32),
            pltpu.SemaphoreType.DMA,
        ],
    )
    def gather_kernel(idx_hbm, table_hbm, out_hbm, idx_v, rows_v, sem):
        wid = lax.axis_index("s") * NC + lax.axis_index("c")
        base = wid * bpw
        pltpu.sync_copy(idx_hbm.at[wid], idx_v)

        def chunk(j):
            for k in range(CH // 16):
                vec = idx_v[j, pl.ds(k * 16, 16)]
                for l in range(16):
                    pltpu.make_async_copy(
                        table_hbm.at[pl.ds(vec[l], 1)],
                        rows_v.at[pl.ds(k * 16 + l, 1)],
                        sem,
                    ).start()
            pltpu.make_async_copy(
                table_hbm.at[pl.ds(0, CH)], rows_v, sem
            ).wait()
            pltpu.sync_copy(rows_v, out_hbm.at[pl.ds(base + j * CH, CH)])

        pl.loop(0, nch)(chunk)

    return gather_kernel(idx3, table)


def _lstm_body(emb_ref, h0_ref, c0_ref, w_ref, u_ref, b_ref,
               out_ref, hf_ref, cf_ref):
    t = pl.program_id(0)

    @pl.when(t == 0)
    def _():
        hf_ref[...] = h0_ref[...]
        cf_ref[...] = c0_ref[...]

    x = emb_ref[0]
    h = hf_ref[...]
    c = cf_ref[...]
    z = (jnp.dot(x, w_ref[...], preferred_element_type=jnp.float32)
         + jnp.dot(h, u_ref[...], preferred_element_type=jnp.float32)
         + b_ref[...])
    i = jax.nn.sigmoid(z[:, 0:H])
    f = jax.nn.sigmoid(z[:, H:2 * H])
    g = jnp.tanh(z[:, 2 * H:3 * H])
    o = jax.nn.sigmoid(z[:, 3 * H:4 * H])
    c_new = f * c + i * g
    h_new = o * jnp.tanh(c_new)
    hf_ref[...] = h_new
    cf_ref[...] = c_new
    out_ref[0] = h_new


def _tc_lstm(embed, state_h, state_c, w, u, b2):
    nt = embed.shape[0]
    out = pl.pallas_call(
        _lstm_body,
        grid=(nt,),
        in_specs=[
            pl.BlockSpec((1, B, D), lambda t: (t, 0, 0)),
            pl.BlockSpec((B, H), lambda t: (0, 0)),
            pl.BlockSpec((B, H), lambda t: (0, 0)),
            pl.BlockSpec((D, 4 * H), lambda t: (0, 0)),
            pl.BlockSpec((H, 4 * H), lambda t: (0, 0)),
            pl.BlockSpec((1, 4 * H), lambda t: (0, 0)),
        ],
        out_specs=[
            pl.BlockSpec((1, B, H), lambda t: (t, 0, 0)),
            pl.BlockSpec((B, H), lambda t: (0, 0)),
            pl.BlockSpec((B, H), lambda t: (0, 0)),
        ],
        out_shape=[
            jax.ShapeDtypeStruct((nt, B, H), jnp.float32),
            jax.ShapeDtypeStruct((B, H), jnp.float32),
            jax.ShapeDtypeStruct((B, H), jnp.float32),
        ],
    )(embed, state_h, state_c, w, u, b2)
    return out[0], out[1], out[2]


T1 = 25
T2 = T - T1


def kernel(sequence, state_h, state_c, embedding, W, U, b):
    idx_t = sequence.astype(jnp.int32).T
    b2 = b.reshape(1, 4 * H)
    e1 = _sc_gather(embedding, idx_t[:T1].reshape(NW, -1, CH))
    e2 = _sc_gather(embedding, idx_t[T1:].reshape(NW, -1, CH))
    o1, h1, c1 = _tc_lstm(e1.reshape(T1, B, D), state_h, state_c, W, U, b2)
    o2, h_fin, c_fin = _tc_lstm(e2.reshape(T2, B, D), h1, c1, W, U, b2)
    out_t = jnp.concatenate([o1, o2], axis=0)
    return (out_t.transpose(1, 0, 2), h_fin, c_fin)


# speedup vs baseline: 1.2308x; 1.0401x over previous
import functools

import jax
import jax.numpy as jnp
from jax import lax
from jax.experimental import pallas as pl
from jax.experimental.pallas import tpu as pltpu
from jax.experimental.pallas import tpu_sc as plsc

V = 1000000
D = 64
H = 128
B = 1024
T = 50

NC = 2
NS = 16
NW = NC * NS
N = B * T
BPW = N // NW
CH = 80
NCH = BPW // CH


def _sc_gather(table, idx3):
    mesh = plsc.VectorSubcoreMesh(core_axis_name="c", subcore_axis_name="s")

    @functools.partial(
        pl.kernel,
        mesh=mesh,
        out_type=jax.ShapeDtypeStruct((N, D), jnp.float32),
        scratch_types=[
            pltpu.VMEM((NCH, CH), jnp.int32),
            pltpu.VMEM((CH, D), jnp.float32),
            pltpu.SemaphoreType.DMA,
        ],
    )
    def gather_kernel(idx_hbm, table_hbm, out_hbm, idx_v, rows_v, sem):
        wid = lax.axis_index("s") * NC + lax.axis_index("c")
        base = wid * BPW
        pltpu.sync_copy(idx_hbm.at[wid], idx_v)

        def chunk(j):
            for k in range(CH // 16):
                vec = idx_v[j, pl.ds(k * 16, 16)]
                for l in range(16):
                    pltpu.make_async_copy(
                        table_hbm.at[pl.ds(vec[l], 1)],
                        rows_v.at[pl.ds(k * 16 + l, 1)],
                        sem,
                    ).start()
            pltpu.make_async_copy(
                table_hbm.at[pl.ds(0, CH)], rows_v, sem
            ).wait()
            pltpu.sync_copy(rows_v, out_hbm.at[pl.ds(base + j * CH, CH)])

        pl.loop(0, NCH)(chunk)

    return gather_kernel(idx3, table)


TS = 2


def _lstm_body(emb_ref, h0_ref, c0_ref, w_ref, u_ref, b_ref,
               out_ref, hf_ref, cf_ref):
    t = pl.program_id(0)

    @pl.when(t == 0)
    def _():
        hf_ref[...] = h0_ref[...]
        cf_ref[...] = c0_ref[...]

    xw = jnp.dot(emb_ref[...].reshape(TS * B, D), w_ref[...],
                 preferred_element_type=jnp.float32)
    h = hf_ref[...]
    c = cf_ref[...]
    for s in range(TS):
        z = (xw[s * B:(s + 1) * B]
             + jnp.dot(h, u_ref[...], preferred_element_type=jnp.float32)
             + b_ref[...])
        i = jax.nn.sigmoid(z[:, 0:H])
        f = jax.nn.sigmoid(z[:, H:2 * H])
        g = jnp.tanh(z[:, 2 * H:3 * H])
        o = jax.nn.sigmoid(z[:, 3 * H:4 * H])
        c = f * c + i * g
        h = o * jnp.tanh(c)
        out_ref[s] = h
    hf_ref[...] = h
    cf_ref[...] = c


def _tc_lstm(embed, state_h, state_c, w, u, b2):
    out = pl.pallas_call(
        _lstm_body,
        grid=(T // TS,),
        in_specs=[
            pl.BlockSpec((TS, B, D), lambda t: (t, 0, 0)),
            pl.BlockSpec((B, H), lambda t: (0, 0)),
            pl.BlockSpec((B, H), lambda t: (0, 0)),
            pl.BlockSpec((D, 4 * H), lambda t: (0, 0)),
            pl.BlockSpec((H, 4 * H), lambda t: (0, 0)),
            pl.BlockSpec((1, 4 * H), lambda t: (0, 0)),
        ],
        out_specs=[
            pl.BlockSpec((TS, B, H), lambda t: (t, 0, 0)),
            pl.BlockSpec((B, H), lambda t: (0, 0)),
            pl.BlockSpec((B, H), lambda t: (0, 0)),
        ],
        out_shape=[
            jax.ShapeDtypeStruct((T, B, H), jnp.float32),
            jax.ShapeDtypeStruct((B, H), jnp.float32),
            jax.ShapeDtypeStruct((B, H), jnp.float32),
        ],
    )(embed, state_h, state_c, w, u, b2)
    return out[0], out[1], out[2]


def kernel(sequence, state_h, state_c, embedding, W, U, b):
    idx3 = sequence.astype(jnp.int32).T.reshape(NW, NCH, CH)
    embed = _sc_gather(embedding, idx3).reshape(T, B, D)
    out_t, h_fin, c_fin = _tc_lstm(embed, state_h, state_c, W, U,
                                   b.reshape(1, 4 * H))
    return (out_t.transpose(1, 0, 2), h_fin, c_fin)


# speedup vs baseline: 1.2360x; 1.0043x over previous
import functools

import jax
import jax.numpy as jnp
from jax import lax
from jax.experimental import pallas as pl
from jax.experimental.pallas import tpu as pltpu
from jax.experimental.pallas import tpu_sc as plsc

V = 1000000
D = 64
H = 128
B = 1024
T = 50

NC = 2
NS = 16
NW = NC * NS
N = B * T
BPW = N // NW
CH = 80
NCH = BPW // CH


def _sc_gather(table, idx3):
    mesh = plsc.VectorSubcoreMesh(core_axis_name="c", subcore_axis_name="s")

    @functools.partial(
        pl.kernel,
        mesh=mesh,
        out_type=jax.ShapeDtypeStruct((N, D), jnp.float32),
        scratch_types=[
            pltpu.VMEM((NCH, CH), jnp.int32),
            pltpu.VMEM((CH, D), jnp.float32),
            pltpu.SemaphoreType.DMA,
        ],
    )
    def gather_kernel(idx_hbm, table_hbm, out_hbm, idx_v, rows_v, sem):
        wid = lax.axis_index("s") * NC + lax.axis_index("c")
        base = wid * BPW
        pltpu.sync_copy(idx_hbm.at[wid], idx_v)

        def chunk(j):
            for k in range(CH // 16):
                vec = idx_v[j, pl.ds(k * 16, 16)]
                for l in range(16):
                    pltpu.make_async_copy(
                        table_hbm.at[pl.ds(vec[l], 1)],
                        rows_v.at[pl.ds(k * 16 + l, 1)],
                        sem,
                    ).start()
            pltpu.make_async_copy(
                table_hbm.at[pl.ds(0, CH)], rows_v, sem
            ).wait()
            pltpu.sync_copy(rows_v, out_hbm.at[pl.ds(base + j * CH, CH)])

        pl.loop(0, NCH)(chunk)

    return gather_kernel(idx3, table)


TS = 5


def _lstm_body(emb_ref, h0_ref, c0_ref, w_ref, u_ref, b_ref,
               out_ref, hf_ref, cf_ref):
    t = pl.program_id(0)

    @pl.when(t == 0)
    def _():
        hf_ref[...] = h0_ref[...]
        cf_ref[...] = c0_ref[...]

    xw = jnp.dot(emb_ref[...].reshape(TS * B, D), w_ref[...],
                 preferred_element_type=jnp.float32)
    h = hf_ref[...]
    c = cf_ref[...]
    for s in range(TS):
        z = (xw[s * B:(s + 1) * B]
             + jnp.dot(h, u_ref[...], preferred_element_type=jnp.float32)
             + b_ref[...])
        i = jax.nn.sigmoid(z[:, 0:H])
        f = jax.nn.sigmoid(z[:, H:2 * H])
        g = jnp.tanh(z[:, 2 * H:3 * H])
        o = jax.nn.sigmoid(z[:, 3 * H:4 * H])
        c = f * c + i * g
        h = o * jnp.tanh(c)
        out_ref[s] = h
    hf_ref[...] = h
    cf_ref[...] = c


def _tc_lstm(embed, state_h, state_c, w, u, b2):
    out = pl.pallas_call(
        _lstm_body,
        grid=(T // TS,),
        in_specs=[
            pl.BlockSpec((TS, B, D), lambda t: (t, 0, 0)),
            pl.BlockSpec((B, H), lambda t: (0, 0)),
            pl.BlockSpec((B, H), lambda t: (0, 0)),
            pl.BlockSpec((D, 4 * H), lambda t: (0, 0)),
            pl.BlockSpec((H, 4 * H), lambda t: (0, 0)),
            pl.BlockSpec((1, 4 * H), lambda t: (0, 0)),
        ],
        out_specs=[
            pl.BlockSpec((TS, B, H), lambda t: (t, 0, 0)),
            pl.BlockSpec((B, H), lambda t: (0, 0)),
            pl.BlockSpec((B, H), lambda t: (0, 0)),
        ],
        out_shape=[
            jax.ShapeDtypeStruct((T, B, H), jnp.float32),
            jax.ShapeDtypeStruct((B, H), jnp.float32),
            jax.ShapeDtypeStruct((B, H), jnp.float32),
        ],
    )(embed, state_h, state_c, w, u, b2)
    return out[0], out[1], out[2]


def kernel(sequence, state_h, state_c, embedding, W, U, b):
    idx3 = sequence.astype(jnp.int32).T.reshape(NW, NCH, CH)
    embed = _sc_gather(embedding, idx3).reshape(T, B, D)
    out_t, h_fin, c_fin = _tc_lstm(embed, state_h, state_c, W, U,
                                   b.reshape(1, 4 * H))
    return (out_t.transpose(1, 0, 2), h_fin, c_fin)


# speedup vs baseline: 1.2371x; 1.0009x over previous
import functools

import jax
import jax.numpy as jnp
from jax import lax
from jax.experimental import pallas as pl
from jax.experimental.pallas import tpu as pltpu
from jax.experimental.pallas import tpu_sc as plsc

V = 1000000
D = 64
H = 128
B = 1024
T = 50

NC = 2
NS = 16
NW = NC * NS
N = B * T
BPW = N // NW
CH = 80
NCH = BPW // CH


def _sc_gather(table, idx3):
    mesh = plsc.VectorSubcoreMesh(core_axis_name="c", subcore_axis_name="s")

    @functools.partial(
        pl.kernel,
        mesh=mesh,
        out_type=jax.ShapeDtypeStruct((N, D), jnp.float32),
        scratch_types=[
            pltpu.VMEM((NCH, CH), jnp.int32),
            pltpu.VMEM((CH, D), jnp.float32),
            pltpu.SemaphoreType.DMA,
        ],
    )
    def gather_kernel(idx_hbm, table_hbm, out_hbm, idx_v, rows_v, sem):
        wid = lax.axis_index("s") * NC + lax.axis_index("c")
        base = wid * BPW
        pltpu.sync_copy(idx_hbm.at[wid], idx_v)

        def chunk(j):
            for k in range(CH // 16):
                vec = idx_v[j, pl.ds(k * 16, 16)]
                for l in range(16):
                    pltpu.make_async_copy(
                        table_hbm.at[pl.ds(vec[l], 1)],
                        rows_v.at[pl.ds(k * 16 + l, 1)],
                        sem,
                    ).start()
            pltpu.make_async_copy(
                table_hbm.at[pl.ds(0, CH)], rows_v, sem
            ).wait()
            pltpu.sync_copy(rows_v, out_hbm.at[pl.ds(base + j * CH, CH)])

        pl.loop(0, NCH)(chunk)

    return gather_kernel(idx3, table)


TS = 10


def _lstm_body(emb_ref, h0_ref, c0_ref, w_ref, u_ref, b_ref,
               out_ref, hf_ref, cf_ref):
    t = pl.program_id(0)

    @pl.when(t == 0)
    def _():
        hf_ref[...] = h0_ref[...]
        cf_ref[...] = c0_ref[...]

    xw = jnp.dot(emb_ref[...].reshape(TS * B, D), w_ref[...],
                 preferred_element_type=jnp.float32)
    h = hf_ref[...]
    c = cf_ref[...]
    for s in range(TS):
        z = (xw[s * B:(s + 1) * B]
             + jnp.dot(h, u_ref[...], preferred_element_type=jnp.float32)
             + b_ref[...])
        i = jax.nn.sigmoid(z[:, 0:H])
        f = jax.nn.sigmoid(z[:, H:2 * H])
        g = jnp.tanh(z[:, 2 * H:3 * H])
        o = jax.nn.sigmoid(z[:, 3 * H:4 * H])
        c = f * c + i * g
        h = o * jnp.tanh(c)
        out_ref[s] = h
    hf_ref[...] = h
    cf_ref[...] = c


def _tc_lstm(embed, state_h, state_c, w, u, b2):
    out = pl.pallas_call(
        _lstm_body,
        grid=(T // TS,),
        in_specs=[
            pl.BlockSpec((TS, B, D), lambda t: (t, 0, 0)),
            pl.BlockSpec((B, H), lambda t: (0, 0)),
            pl.BlockSpec((B, H), lambda t: (0, 0)),
            pl.BlockSpec((D, 4 * H), lambda t: (0, 0)),
            pl.BlockSpec((H, 4 * H), lambda t: (0, 0)),
            pl.BlockSpec((1, 4 * H), lambda t: (0, 0)),
        ],
        out_specs=[
            pl.BlockSpec((TS, B, H), lambda t: (t, 0, 0)),
            pl.BlockSpec((B, H), lambda t: (0, 0)),
            pl.BlockSpec((B, H), lambda t: (0, 0)),
        ],
        out_shape=[
            jax.ShapeDtypeStruct((T, B, H), jnp.float32),
            jax.ShapeDtypeStruct((B, H), jnp.float32),
            jax.ShapeDtypeStruct((B, H), jnp.float32),
        ],
    )(embed, state_h, state_c, w, u, b2)
    return out[0], out[1], out[2]


def kernel(sequence, state_h, state_c, embedding, W, U, b):
    idx3 = sequence.astype(jnp.int32).T.reshape(NW, NCH, CH)
    embed = _sc_gather(embedding, idx3).reshape(T, B, D)
    out_t, h_fin, c_fin = _tc_lstm(embed, state_h, state_c, W, U,
                                   b.reshape(1, 4 * H))
    return (out_t.transpose(1, 0, 2), h_fin, c_fin)
